# Initial kernel scaffold; baseline (speedup 1.0000x reference)
import jax
import jax.numpy as jnp
from jax.experimental import pallas as pl


def kernel(hidden, gt_texts, gt_kernels, training_masks, gt_instances):
    raise NotImplementedError("write your pallas kernel here")


# speedup vs baseline: 79.2204x; 79.2204x over previous
import jax
import jax.numpy as jnp
from jax import lax
from jax.experimental import pallas as pl
from jax.experimental.pallas import tpu as pltpu

_B, _H, _W = 8, 512, 512
_FD = 4
_K = 8
_NEG_INF = float("-inf")
_INT_MIN = -2147483648


def _pool1d(x, axis, idx, n):
    r = x
    for s in range(1, 5):
        up = pltpu.roll(x, n - s, axis)
        up = jnp.where(idx < n - s, up, _NEG_INF)
        dn = pltpu.roll(x, s, axis)
        dn = jnp.where(idx >= s, dn, _NEG_INF)
        r = jnp.maximum(r, jnp.maximum(up, dn))
    return r


def _dice(inp, target, mask):
    i = (1.0 / (1.0 + jnp.exp(-inp))) * mask
    t = target * mask
    a = jnp.sum(i * t)
    bb = jnp.sum(i * i) + 0.001
    c = jnp.sum(t * t) + 0.001
    return 0.5 * (1.0 - 2.0 * a / (bb + c))


def _body(hid_ref, gt_ref, gk_ref, tm_ref, inst_ref, out_ref):
    imin = jnp.int32(_INT_MIN)
    ker = hid_ref[0, 0]
    gt = gt_ref[0]
    gk = gk_ref[0]
    tm = tm_ref[0]
    inst = inst_ref[0]

    row_idx = lax.broadcasted_iota(jnp.int32, (_H, _W), 0)
    col_idx = lax.broadcasted_iota(jnp.int32, (_H, _W), 1)

    texts = _pool1d(_pool1d(ker, 1, col_idx, _W), 0, row_idx, _H)

    pos = gt > 0.5
    tm_pos = tm > 0.5
    pos_num = jnp.sum(jnp.where(pos & tm_pos, 1.0, 0.0))
    neg_count = jnp.float32(_H * _W) - jnp.sum(jnp.where(pos, 1.0, 0.0))
    neg_num = jnp.minimum(3.0 * pos_num, neg_count)
    k_f = jnp.maximum(neg_num, 1.0)

    sbits = lax.bitcast_convert_type(texts, jnp.int32)
    key = jnp.where(sbits >= 0, sbits, imin - sbits)
    mk = jnp.where(pos, imin, key)

    t_bits = jnp.int32(0)
    for b in range(31, -1, -1):
        bit = imin if b == 31 else jnp.int32(1 << b)
        cand = t_bits | bit
        scand = cand ^ imin
        cnt = jnp.sum(jnp.where(mk >= scand, 1.0, 0.0))
        t_bits = jnp.where(cnt >= k_f, cand, t_bits)
    sthr = t_bits ^ imin

    sel = jnp.where(((key >= sthr) | pos) & tm_pos, 1.0, 0.0)
    selected = jnp.where((pos_num == 0.0) | (neg_num == 0.0), tm, sel)

    loss_text = _dice(texts, gt, selected)
    loss_kernel = _dice(ker, gk, gt * tm)

    keri = gk > 0.5
    inst2d = jnp.where(tm_pos, inst, 0)
    instk = jnp.where(keri, inst2d, 0)
    embs = [hid_ref[0, 1 + c] for c in range(_FD)]

    means = [[None] * _FD for _ in range(_K)]
    for kk in range(_K):
        if kk == 0:
            for c in range(_FD):
                means[0][c] = jnp.float32(0.0)
            continue
        mseg = jnp.where(instk == kk, 1.0, 0.0)
        denom = jnp.maximum(jnp.sum(mseg), 1.0)
        for c in range(_FD):
            means[kk][c] = jnp.sum(mseg * embs[c]) / denom

    mean_maps = [jnp.zeros((_H, _W), jnp.float32) for _ in range(_FD)]
    for kk in range(1, _K):
        minst = inst2d == kk
        for c in range(_FD):
            mean_maps[c] = jnp.where(minst, means[kk][c], mean_maps[c])

    dist2 = jnp.float32(1e-12)
    for c in range(_FD):
        d = embs[c] - mean_maps[c]
        dist2 = dist2 + d * d
    dist = jnp.sqrt(dist2)
    rel = jnp.maximum(dist - 0.5, 0.0)
    v = jnp.log(rel * rel + 1.0)

    l_agg = jnp.float32(0.0)
    for kk in range(1, _K):
        m2 = jnp.where(inst2d == kk, 1.0, 0.0)
        l_agg = l_agg + jnp.sum(m2 * v) / jnp.maximum(jnp.sum(m2), 1.0)
    l_agg = l_agg / (_K - 1)

    ii = lax.broadcasted_iota(jnp.int32, (_K, 128), 0)
    jj = lax.broadcasted_iota(jnp.int32, (_K, 128), 1)

    def bcast_rows(vals, idx):
        r = jnp.zeros((_K, 128), jnp.float32)
        for kk in range(_K):
            r = jnp.where(idx == kk, vals[kk], r)
        return r

    pd2 = jnp.float32(1e-12)
    n2 = jnp.float32(1e-12)
    for c in range(_FD):
        mrow = bcast_rows([means[kk][c] for kk in range(_K)], ii)
        mcol = bcast_rows([means[kk][c] for kk in range(_K)], jj)
        dd = mrow - mcol
        pd2 = pd2 + dd * dd
        n2 = n2 + mrow * mrow
    pd = jnp.sqrt(pd2)
    rr = jnp.maximum(2.0 * 1.5 - pd, 0.0)
    dmat = jnp.log(rr * rr + 1.0)
    mvalid = jnp.where(
        (ii >= 1) & (jj >= 1) & (jj < _K) & (ii != jj), 1.0, 0.0
    )
    l_dis = jnp.sum(dmat * mvalid) / 42.0
    regv = jnp.log(jnp.sqrt(n2) + 1.0)
    l_reg = jnp.sum(jnp.where(jj == 0, regv, 0.0)) / _K * 0.001

    loss_emb = l_agg + l_dis + l_reg

    lane = lax.broadcasted_iota(jnp.int32, (1, 128), 1)
    row = (
        jnp.where(lane == 0, loss_text, 0.0)
        + jnp.where(lane == 1, loss_kernel, 0.0)
        + jnp.where(lane == 2, loss_emb, 0.0)
    )
    out_ref[0] = row


def _call(hidden, gt_texts, gt_kernels, training_masks, gt_instances,
          interpret=False):
    return pl.pallas_call(
        _body,
        grid=(_B,),
        in_specs=[
            pl.BlockSpec((1, 1 + _FD, _H, _W), lambda i: (i, 0, 0, 0)),
            pl.BlockSpec((1, _H, _W), lambda i: (i, 0, 0)),
            pl.BlockSpec((1, _H, _W), lambda i: (i, 0, 0)),
            pl.BlockSpec((1, _H, _W), lambda i: (i, 0, 0)),
            pl.BlockSpec((1, _H, _W), lambda i: (i, 0, 0)),
        ],
        out_specs=pl.BlockSpec((1, 1, 128), lambda i: (i, 0, 0)),
        out_shape=jax.ShapeDtypeStruct((_B, 1, 128), jnp.float32),
        compiler_params=pltpu.CompilerParams(
            dimension_semantics=("arbitrary",),
        ),
        interpret=interpret,
    )(hidden, gt_texts, gt_kernels, training_masks, gt_instances)


def kernel(hidden, gt_texts, gt_kernels, training_masks, gt_instances):
    out = _call(hidden, gt_texts, gt_kernels, training_masks, gt_instances)
    loss_text = out[:, 0, 0]
    loss_kernel = out[:, 0, 1]
    loss_emb = 0.25 * out[:, 0, 2]
    return jnp.mean(loss_text) + jnp.mean(loss_kernel) + jnp.mean(loss_emb)
